# baseline (speedup 1.0000x reference)
import functools

import jax
import jax.numpy as jnp
from jax import lax
from jax.experimental import pallas as pl
from jax.experimental.pallas import tpu as pltpu

_KS = 3
_EPS = 0.001


def _lane_shift(a, o, zeros):
    if o == 0:
        return a
    if o > 0:
        return jnp.concatenate([a[:, o:], zeros[:, :o]], axis=1)
    return jnp.concatenate([zeros[:, :(-o)], a[:, :a.shape[1] + o]], axis=1)


def _conv_stats_kernel(x_ref, w_ref, y_ref, stat_ref, stat_acc, *, nb, d, h, w):
    cin = x_ref.shape[1]
    hw = h * w
    dhw = d * hw
    step = pl.program_id(0)

    @pl.when(step == 0)
    def _():
        stat_acc[...] = jnp.zeros_like(stat_acc)

    lane = lax.broadcasted_iota(jnp.int32, (1, dhw), 1)
    w_idx = lane % w
    h_idx = (lane // w) % h
    conds = {}
    for kh in range(_KS):
        for kw in range(_KS):
            cond = None
            if kh != 1:
                hh = h_idx + (kh - 1)
                cond = (hh >= 0) if kh == 0 else (hh < h)
            if kw != 1:
                ww = w_idx + (kw - 1)
                c = (ww >= 0) if kw == 0 else (ww < w)
                cond = c if cond is None else jnp.logical_and(cond, c)
            conds[(kh, kw)] = cond

    stat = stat_acc[...]
    for b in range(nb):
        xi = x_ref[b].reshape(cin, dhw).astype(jnp.bfloat16)
        zeros = jnp.zeros_like(xi)

        xhw = {}
        for kh in range(_KS):
            for kw in range(_KS):
                o = (kh - 1) * w + (kw - 1)
                s = _lane_shift(xi, o, zeros)
                if conds[(kh, kw)] is not None:
                    s = jnp.where(conds[(kh, kw)], s, jnp.bfloat16(0))
                xhw[(kh, kw)] = s

        taps = []
        for kd in range(_KS):
            o = (kd - 1) * hw
            for kh in range(_KS):
                for kw in range(_KS):
                    taps.append(_lane_shift(xhw[(kh, kw)], o, zeros))
        patches = jnp.concatenate(taps, axis=0)

        acc = jnp.dot(w_ref[...], patches,
                      preferred_element_type=jnp.float32)
        y_ref[b] = acc.astype(y_ref.dtype)

        s1 = jnp.sum(acc, axis=1, keepdims=True)
        s2 = jnp.sum(acc * acc, axis=1, keepdims=True)
        stat = stat + jnp.concatenate([s1, s2], axis=1)
    stat_acc[...] = stat
    stat_ref[...] = stat_acc[...]


def _affine_kernel(y_ref, stat_ref, gam_ref, bet_ref, o_ref, *, inv_cnt):
    st = stat_ref[...]
    mean = st[:, 0:1] * inv_cnt
    var = st[:, 1:2] * inv_cnt - mean * mean
    scale = gam_ref[...] * lax.rsqrt(var + _EPS)
    shift = bet_ref[...] - mean * scale
    y = y_ref[...].astype(jnp.float32)
    o_ref[...] = y * scale[None] + shift[None]


@jax.jit
def kernel(x_ncdhw, w2d, gamma, beta):
    n, cin, d, h, w = x_ncdhw.shape
    cout = w2d.shape[0]
    dhw = d * h * w

    w_bf = w2d.astype(jnp.bfloat16)

    nb = 2 if n % 2 == 0 else 1
    grid1 = n // nb

    y, stats = pl.pallas_call(
        functools.partial(_conv_stats_kernel, nb=nb, d=d, h=h, w=w),
        out_shape=(
            jax.ShapeDtypeStruct((n, cout, dhw), jnp.bfloat16),
            jax.ShapeDtypeStruct((cout, 2), jnp.float32),
        ),
        grid=(grid1,),
        in_specs=[
            pl.BlockSpec((nb, cin, d, h, w), lambda i: (i, 0, 0, 0, 0)),
            pl.BlockSpec((cout, w2d.shape[1]), lambda i: (0, 0)),
        ],
        out_specs=(
            pl.BlockSpec((nb, cout, dhw), lambda i: (i, 0, 0)),
            pl.BlockSpec((cout, 2), lambda i: (0, 0)),
        ),
        scratch_shapes=[pltpu.VMEM((cout, 2), jnp.float32)],
        compiler_params=pltpu.CompilerParams(
            dimension_semantics=("arbitrary",),
        ),
    )(x_ncdhw, w_bf)

    nb2 = 8 if n % 8 == 0 else (2 if n % 2 == 0 else 1)
    grid2 = n // nb2

    out = pl.pallas_call(
        functools.partial(_affine_kernel, inv_cnt=1.0 / float(n * dhw)),
        out_shape=jax.ShapeDtypeStruct((n, cout, dhw), jnp.float32),
        grid=(grid2,),
        in_specs=[
            pl.BlockSpec((nb2, cout, dhw), lambda i: (i, 0, 0)),
            pl.BlockSpec((cout, 2), lambda i: (0, 0)),
            pl.BlockSpec((cout, 1), lambda i: (0, 0)),
            pl.BlockSpec((cout, 1), lambda i: (0, 0)),
        ],
        out_specs=pl.BlockSpec((nb2, cout, dhw), lambda i: (i, 0, 0)),
        compiler_params=pltpu.CompilerParams(
            dimension_semantics=("arbitrary",),
        ),
    )(y, stats, gamma.reshape(cout, 1), beta.reshape(cout, 1))

    return out.reshape(n, cout, d, h, w)


# speedup vs baseline: 1.9450x; 1.9450x over previous
import functools

import jax
import jax.numpy as jnp
from jax import lax
from jax.experimental import pallas as pl
from jax.experimental.pallas import tpu as pltpu

_KS = 3
_EPS = 0.001


def _lane_shift(a, o, zeros):
    if o == 0:
        return a
    if o > 0:
        return jnp.concatenate([a[:, o:], zeros[:, :o]], axis=1)
    return jnp.concatenate([zeros[:, :(-o)], a[:, :a.shape[1] + o]], axis=1)


def _fused_kernel(x_ref, w_ref, gam_ref, bet_ref, o_ref,
                  y_buf, stat_acc, *, nb, nb2, d, h, w, n):
    cin = x_ref.shape[1]
    cout = o_ref.shape[1]
    hw = h * w
    dhw = d * hw
    conv_steps = n // nb
    step = pl.program_id(0)

    @pl.when(step == 0)
    def _():
        stat_acc[...] = jnp.zeros_like(stat_acc)

    @pl.when(step < conv_steps)
    def _conv_phase():
        lane = lax.broadcasted_iota(jnp.int32, (1, dhw), 1)
        w_idx = lane % w
        h_idx = (lane // w) % h
        conds = {}
        for kh in range(_KS):
            for kw in range(_KS):
                cond = None
                if kh != 1:
                    hh = h_idx + (kh - 1)
                    cond = (hh >= 0) if kh == 0 else (hh < h)
                if kw != 1:
                    ww = w_idx + (kw - 1)
                    c = (ww >= 0) if kw == 0 else (ww < w)
                    cond = c if cond is None else jnp.logical_and(cond, c)
                conds[(kh, kw)] = cond

        stat = stat_acc[...]
        for b in range(nb):
            xi = x_ref[b].astype(jnp.bfloat16)
            zeros = jnp.zeros_like(xi)

            xhw = {}
            for kh in range(_KS):
                for kw in range(_KS):
                    o = (kh - 1) * w + (kw - 1)
                    s = _lane_shift(xi, o, zeros)
                    if conds[(kh, kw)] is not None:
                        s = jnp.where(conds[(kh, kw)], s, jnp.bfloat16(0))
                    xhw[(kh, kw)] = s

            taps = []
            for kd in range(_KS):
                o = (kd - 1) * hw
                for kh in range(_KS):
                    for kw in range(_KS):
                        taps.append(_lane_shift(xhw[(kh, kw)], o, zeros))
            patches = jnp.concatenate(taps, axis=0)

            acc = jnp.dot(w_ref[...], patches,
                          preferred_element_type=jnp.float32)
            y_buf[step * nb + b] = acc.astype(y_buf.dtype)

            s1 = jnp.sum(acc, axis=1, keepdims=True)
            s2 = jnp.sum(acc * acc, axis=1, keepdims=True)
            stat = stat + jnp.concatenate([s1, s2], axis=1)
        stat_acc[...] = stat

    @pl.when(step >= conv_steps)
    def _affine_phase():
        st = stat_acc[...]
        inv_cnt = 1.0 / float(n * dhw)
        mean = st[:, 0:1] * inv_cnt
        var = st[:, 1:2] * inv_cnt - mean * mean
        scale = gam_ref[...] * lax.rsqrt(var + _EPS)
        shift = bet_ref[...] - mean * scale
        j = step - conv_steps
        y = y_buf[pl.ds(j * nb2, nb2)].astype(jnp.float32)
        o_ref[...] = y * scale[None] + shift[None]


@jax.jit
def kernel(x_ncdhw, w2d, gamma, beta):
    n, cin, d, h, w = x_ncdhw.shape
    cout = w2d.shape[0]
    dhw = d * h * w

    x_flat = x_ncdhw.reshape(n, cin, dhw)
    w_bf = w2d.astype(jnp.bfloat16)

    nb = 4 if n % 4 == 0 else 1
    nb2 = 4 if n % 4 == 0 else 1
    conv_steps = n // nb
    affine_steps = n // nb2
    grid = conv_steps + affine_steps

    out = pl.pallas_call(
        functools.partial(_fused_kernel, nb=nb, nb2=nb2, d=d, h=h, w=w, n=n),
        out_shape=jax.ShapeDtypeStruct((n, cout, dhw), jnp.float32),
        grid=(grid,),
        in_specs=[
            pl.BlockSpec((nb, cin, dhw),
                         lambda i: (jnp.minimum(i, conv_steps - 1), 0, 0)),
            pl.BlockSpec((cout, w2d.shape[1]), lambda i: (0, 0)),
            pl.BlockSpec((cout, 1), lambda i: (0, 0)),
            pl.BlockSpec((cout, 1), lambda i: (0, 0)),
        ],
        out_specs=pl.BlockSpec(
            (nb2, cout, dhw),
            lambda i: (jnp.maximum(i - conv_steps, 0), 0, 0)),
        scratch_shapes=[
            pltpu.VMEM((n, cout, dhw), jnp.bfloat16),
            pltpu.VMEM((cout, 2), jnp.float32),
        ],
        compiler_params=pltpu.CompilerParams(
            dimension_semantics=("arbitrary",),
        ),
    )(x_flat, w_bf, gamma.reshape(cout, 1), beta.reshape(cout, 1))

    return out.reshape(n, cout, d, h, w)
